# Initial kernel scaffold; baseline (speedup 1.0000x reference)
import jax
import jax.numpy as jnp
from jax.experimental import pallas as pl


def kernel(z, embedding):
    raise NotImplementedError("write your pallas kernel here")


# speedup vs baseline: 1.1543x; 1.1543x over previous
import functools

import jax
import jax.numpy as jnp
from jax import lax
from jax.experimental import pallas as pl
from jax.experimental.pallas import tpu as pltpu
from jax.experimental.pallas import tpu_sc as plsc

N_CODES = 8192
D = 256
LB = 512
KB = 4096

NC, NS, NL = 2, 16, 16
NW = NC * NS
CH = 128


def _argmin_body(zf_ref, emb_ref, rs_ref, cs_ref, idx_ref, bestv_ref, besti_ref):
    k = pl.program_id(0)
    l = pl.program_id(1)
    zb = zf_ref[...]
    eb = emb_ref[...]
    mm = lax.dot_general(zb.astype(jnp.bfloat16), eb.astype(jnp.bfloat16),
                         (((1,), (1,)), ((), ())),
                         preferred_element_type=jnp.float32)
    rs = rs_ref[...]
    cs = cs_ref[...]
    d = (rs + cs) - 2.0 * mm
    bmin = jnp.min(d, axis=1, keepdims=True)
    kidx = lax.broadcasted_iota(jnp.int32, (LB, KB), 1) + k * KB
    bidx = jnp.min(jnp.where(d == bmin, kidx, jnp.int32(2**30)),
                   axis=1, keepdims=True)

    lsl = pl.ds(l * LB, LB)

    @pl.when(k == 0)
    def _():
        bestv_ref[lsl, :] = bmin
        besti_ref[lsl, :] = bidx

    @pl.when(k > 0)
    def _():
        prevv = bestv_ref[lsl, :]
        previ = besti_ref[lsl, :]
        upd = bmin < prevv
        bestv_ref[lsl, :] = jnp.where(upd, bmin, prevv)
        besti_ref[lsl, :] = jnp.where(upd, bidx, previ)

    idx_ref[...] = besti_ref[lsl, :]


def _tc_argmin(zf, embedding, rs, cs):
    L = zf.shape[0]
    nk = N_CODES // KB
    nl = L // LB
    idx2d = pl.pallas_call(
        _argmin_body,
        grid=(nk, nl),
        in_specs=[
            pl.BlockSpec((LB, D), lambda k, l: (l, 0)),
            pl.BlockSpec((KB, D), lambda k, l: (k, 0)),
            pl.BlockSpec((LB, 1), lambda k, l: (l, 0)),
            pl.BlockSpec((1, KB), lambda k, l: (0, k)),
        ],
        out_specs=pl.BlockSpec((LB, 1), lambda k, l: (l, 0)),
        out_shape=jax.ShapeDtypeStruct((L, 1), jnp.int32),
        scratch_shapes=[
            pltpu.VMEM((L, 1), jnp.float32),
            pltpu.VMEM((L, 1), jnp.int32),
        ],
    )(zf, embedding, rs, cs)
    return idx2d.reshape(-1)


def _sc_body(emb_hbm, idx_hbm, zf_hbm, zq_hbm, loss_hbm, cnt_hbm,
             idx_v, rows_v, zfc_v, pres_v, idxall_v, out16f_v, out16i_v, sem):
    L = NW * 4 * CH
    c = lax.axis_index("c")
    s = lax.axis_index("s")
    w = s * NC + c
    rpw = L // NW
    base = w * rpw

    acc = jnp.zeros((NL,), jnp.float32)
    for ch in range(rpw // CH):
        off = base + ch * CH
        pltpu.sync_copy(idx_hbm.at[pl.ds(off, CH)], idx_v)
        pltpu.async_copy(emb_hbm.at[idx_v], rows_v, sem).wait()
        pltpu.sync_copy(zf_hbm.at[pl.ds(off, CH)], zfc_v)
        pltpu.sync_copy(rows_v, zq_hbm.at[pl.ds(off, CH)])

        def loss_step(i, a):
            r = i // (D // NL)
            j = i % (D // NL)
            x = rows_v[r, pl.ds(j * NL, NL)]
            y = zfc_v[r, pl.ds(j * NL, NL)]
            dlt = x - y
            return a + dlt * dlt

        acc = lax.fori_loop(0, CH * (D // NL), loss_step, acc)

    out16f_v[...] = acc
    pltpu.sync_copy(out16f_v, loss_hbm.at[w])

    pltpu.sync_copy(idx_hbm, idxall_v)
    zeros16 = jnp.zeros((NL,), jnp.int32)
    for t in range(256 // NL):
        pres_v[pl.ds(t * NL, NL)] = zeros16

    lo = w * (N_CODES // NW)
    ones16 = jnp.ones((NL,), jnp.int32)

    def scan_step(i, carry):
        v = idxall_v[pl.ds(i * NL, NL)]
        rel = v - lo
        m = (rel >= 0) & (rel < (N_CODES // NW))
        relc = jnp.clip(rel, 0, (N_CODES // NW) - 1)
        plsc.store_scatter(pres_v, [relc], ones16, mask=m)
        return carry

    lax.fori_loop(0, L // NL, scan_step, 0)

    def cnt_step(t, a):
        return a + pres_v[pl.ds(t * NL, NL)]

    cnt = lax.fori_loop(0, 256 // NL, cnt_step, jnp.zeros((NL,), jnp.int32))
    out16i_v[...] = cnt
    pltpu.sync_copy(out16i_v, cnt_hbm.at[w])


def _sc_gather(embedding, idx, zf):
    L = zf.shape[0]
    mesh = plsc.VectorSubcoreMesh(core_axis_name="c", subcore_axis_name="s")
    kern = pl.kernel(
        _sc_body,
        mesh=mesh,
        compiler_params=pltpu.CompilerParams(needs_layout_passes=False),
        out_type=[
            jax.ShapeDtypeStruct((L, D), jnp.float32),
            jax.ShapeDtypeStruct((NW, NL), jnp.float32),
            jax.ShapeDtypeStruct((NW, NL), jnp.int32),
        ],
        scratch_types=[
            pltpu.VMEM((CH,), jnp.int32),
            pltpu.VMEM((CH, D), jnp.float32),
            pltpu.VMEM((CH, D), jnp.float32),
            pltpu.VMEM((N_CODES // NW,), jnp.int32),
            pltpu.VMEM((L,), jnp.int32),
            pltpu.VMEM((NL,), jnp.float32),
            pltpu.VMEM((NL,), jnp.int32),
            pltpu.SemaphoreType.DMA,
        ],
    )
    return kern(embedding, idx, zf)


def _finalize_body(loss_ref, cnt_ref, loss_o, used_o):
    lp = loss_ref[...]
    cp = cnt_ref[...]
    n = jnp.float32(NW * 4 * CH * D)
    loss_o[...] = jnp.full((1, 1), jnp.sum(lp) / n, jnp.float32)
    used_o[...] = jnp.full((1, 1), jnp.sum(cp).astype(jnp.float32), jnp.float32)


def _tc_finalize(loss_parts, cnt_parts):
    return pl.pallas_call(
        _finalize_body,
        out_shape=[
            jax.ShapeDtypeStruct((1, 1), jnp.float32),
            jax.ShapeDtypeStruct((1, 1), jnp.float32),
        ],
    )(loss_parts, cnt_parts)


def kernel(z, embedding):
    B, C, H, W = z.shape
    zp = jnp.transpose(z, (0, 2, 3, 1))
    zf = zp.reshape(-1, D)
    rs = jnp.sum(zf ** 2, axis=1, keepdims=True)
    cs = jnp.sum(embedding ** 2, axis=1).reshape(1, -1)

    min_encoding_indices = _tc_argmin(zf, embedding, rs, cs)

    zq_flat, loss_parts, cnt_parts = _sc_gather(embedding, min_encoding_indices, zf)
    loss2d, used2d = _tc_finalize(loss_parts, cnt_parts)

    z_q = zq_flat.reshape(B, H, W, C)
    z_q = jnp.transpose(z_q, (0, 3, 1, 2))
    loss = loss2d.reshape(())
    used = used2d.reshape(())
    index = min_encoding_indices.reshape(B, H, W)
    return (z_q, used, loss, index)
